# baseline (speedup 1.0000x reference)
import jax
import jax.numpy as jnp
from jax import lax
from jax.experimental import pallas as pl
from jax.experimental.pallas import tpu as pltpu
from jax.experimental.pallas import tpu_sc as plsc

N_NEURONS = 8192
D_STATE = 256
B = 4096

NC = 2
NS = 16
L = 16
NW = NC * NS
BPW = B // NW
G = 4
NSLOT = 2
NG = BPW // G

_mesh = plsc.VectorSubcoreMesh(
    core_axis_name="c", subcore_axis_name="s", num_cores=NC, num_subcores=NS
)


def _body(idx_hbm, idxg_hbm, sig_hbm, conn_hbm, sig_out, conn_out,
          idx_all, my_idx, gidx, rows_buf, out_buf, sig_buf,
          gsem0, gsem1, osem0, osem1, ssem, osig):
  gsems = [gsem0, gsem1]
  osems = [osem0, osem1]
  cid = lax.axis_index("c")
  sid = lax.axis_index("s")
  wid = sid * NC + cid
  base = wid * BPW

  pltpu.sync_copy(idx_hbm, idx_all)
  pltpu.sync_copy(idx_hbm.at[pl.ds(base, BPW)], my_idx.at[pl.ds(0, BPW)])
  pltpu.sync_copy(idxg_hbm.at[wid], gidx)

  SIGC = BPW // 2
  sig_cp = pltpu.async_copy(
      sig_hbm.at[my_idx.at[pl.ds(0, SIGC)]], sig_buf, ssem
  )

  def _rslot(slot):
    return rows_buf.at[slot]

  def _oslot(slot):
    return out_buf.at[slot]

  def _start_group(gi, slot):
    pltpu.async_copy(conn_hbm.at[gidx.at[gi]], _rslot(slot), gsems[slot])

  for slot in range(NSLOT):
    _start_group(slot, slot)

  @pl.loop(0, NG // NSLOT)
  def _grp(rr):
    for slot in range(NSLOT):
      gi = rr * NSLOT + slot
      pltpu.make_async_copy(
          conn_hbm.at[gidx.at[0]], _rslot(slot), gsems[slot]
      ).wait()

      @pl.when(rr > 0)
      def _():
        pltpu.make_async_copy(
            _oslot(slot), conn_out.at[pl.ds(base, G)], osems[slot]
        ).wait()

      @pl.loop(0, B // L)
      def _cols(j):
        cvec = idx_all[pl.ds(j * L, L)]
        for g in range(G):
          gvec = jnp.full((L,), g, jnp.int32)
          out_buf[slot, g, pl.ds(j * L, L)] = plsc.load_gather(
              rows_buf.at[slot], [gvec, cvec]
          )

      @pl.when(gi + NSLOT < NG)
      def _():
        _start_group(gi + NSLOT, slot)

      pltpu.async_copy(
          _oslot(slot), conn_out.at[pl.ds(base + gi * G, G)], osems[slot]
      )

  for slot in range(NSLOT):
    pltpu.make_async_copy(
        _oslot(slot), conn_out.at[pl.ds(base, G)], osems[slot]
    ).wait()

  sig_cp.wait()
  pltpu.async_copy(sig_buf, sig_out.at[pl.ds(base, SIGC)], osig).wait()
  pltpu.async_copy(
      sig_hbm.at[my_idx.at[pl.ds(SIGC, SIGC)]], sig_buf, ssem
  ).wait()
  pltpu.sync_copy(sig_buf, sig_out.at[pl.ds(base + SIGC, SIGC)])


@jax.jit
def _pool(indices, neuron_signatures, connection_strength):
  run = pl.kernel(
      _body,
      out_type=[
          jax.ShapeDtypeStruct((B, D_STATE), jnp.float32),
          jax.ShapeDtypeStruct((B, B), jnp.float32),
      ],
      mesh=_mesh,
      compiler_params=pltpu.CompilerParams(needs_layout_passes=False),
      scratch_types=[
          pltpu.VMEM((B,), jnp.int32),
          pltpu.VMEM((BPW + L,), jnp.int32),
          pltpu.VMEM((NG, G), jnp.int32),
          pltpu.VMEM((NSLOT, G, N_NEURONS), jnp.float32),
          pltpu.VMEM((NSLOT, G, B), jnp.float32),
          pltpu.VMEM((BPW // 2, D_STATE), jnp.float32),
          pltpu.SemaphoreType.DMA,
          pltpu.SemaphoreType.DMA,
          pltpu.SemaphoreType.DMA,
          pltpu.SemaphoreType.DMA,
          pltpu.SemaphoreType.DMA,
          pltpu.SemaphoreType.DMA,
      ],
  )
  idx_grouped = indices.reshape(NW, NG, G)
  sigs, conns = run(
      indices, idx_grouped, neuron_signatures, connection_strength
  )
  return sigs, conns


def kernel(indices, neuron_signatures, connection_strength):
  idx = indices.astype(jnp.int32)
  return _pool(idx, neuron_signatures, connection_strength)


# speedup vs baseline: 1.2077x; 1.2077x over previous
import jax
import jax.numpy as jnp
from jax import lax
from jax.experimental import pallas as pl
from jax.experimental.pallas import tpu as pltpu
from jax.experimental.pallas import tpu_sc as plsc

N_NEURONS = 8192
D_STATE = 256
B = 4096

NC = 2
NS = 16
L = 16
NW = NC * NS
BPW = B // NW
G = 4
NSLOT = 2
NG = BPW // G

_mesh = plsc.VectorSubcoreMesh(
    core_axis_name="c", subcore_axis_name="s", num_cores=NC, num_subcores=NS
)


def _body(idx_hbm, sig_hbm, conn_hbm, sig_out, conn_out,
          idx_all, my_idx, rows_buf, out_buf, sig_buf,
          gsem0, gsem1, osem0, osem1, ssem, osig):
  gsems = [gsem0, gsem1]
  osems = [osem0, osem1]
  cid = lax.axis_index("c")
  sid = lax.axis_index("s")
  wid = sid * NC + cid
  base = wid * BPW

  pltpu.sync_copy(idx_hbm, idx_all)
  pltpu.sync_copy(idx_hbm.at[pl.ds(base, BPW)], my_idx.at[pl.ds(0, BPW)])

  SIGC = BPW // 2
  sig_cp = pltpu.async_copy(
      sig_hbm.at[my_idx.at[pl.ds(0, SIGC)]], sig_buf, ssem
  )

  def _row_src(r):
    vec = my_idx[pl.ds(r, L)]
    return conn_hbm.at[vec[0]]

  dummy_row = conn_hbm.at[0]

  def _rslot(slot, g):
    return rows_buf.at[pl.ds((slot * G + g) * N_NEURONS, N_NEURONS)]

  def _oslot(slot, g):
    return out_buf.at[pl.ds((slot * G + g) * B, B)]

  def _start_group(gi, slot):
    for g in range(G):
      pltpu.async_copy(_row_src(gi * G + g), _rslot(slot, g), gsems[slot])

  for slot in range(NSLOT):
    _start_group(slot, slot)

  @pl.loop(0, NG // NSLOT)
  def _grp(rr):
    for slot in range(NSLOT):
      gi = rr * NSLOT + slot
      for g in range(G):
        pltpu.make_async_copy(dummy_row, _rslot(slot, g), gsems[slot]).wait()

      @pl.when(rr > 0)
      def _():
        for g in range(G):
          pltpu.make_async_copy(
              _oslot(slot, g), conn_out.at[base], osems[slot]
          ).wait()

      @pl.loop(0, B // L, unroll=4)
      def _cols(j):
        cvec = idx_all[pl.ds(j * L, L)]
        for g in range(G):
          out_buf[pl.ds((slot * G + g) * B + j * L, L)] = plsc.load_gather(
              rows_buf, [cvec + jnp.int32((slot * G + g) * N_NEURONS)]
          )

      @pl.when(gi + NSLOT < NG)
      def _():
        _start_group(gi + NSLOT, slot)

      for g in range(G):
        pltpu.async_copy(
            _oslot(slot, g), conn_out.at[base + gi * G + g], osems[slot]
        )

  for slot in range(NSLOT):
    for g in range(G):
      pltpu.make_async_copy(
          _oslot(slot, g), conn_out.at[base], osems[slot]
      ).wait()

  sig_cp.wait()
  pltpu.async_copy(sig_buf, sig_out.at[pl.ds(base, SIGC)], osig).wait()
  pltpu.async_copy(
      sig_hbm.at[my_idx.at[pl.ds(SIGC, SIGC)]], sig_buf, ssem
  ).wait()
  pltpu.sync_copy(sig_buf, sig_out.at[pl.ds(base + SIGC, SIGC)])


@jax.jit
def _pool(indices, neuron_signatures, connection_strength):
  run = pl.kernel(
      _body,
      out_type=[
          jax.ShapeDtypeStruct((B, D_STATE), jnp.float32),
          jax.ShapeDtypeStruct((B, B), jnp.float32),
      ],
      mesh=_mesh,
      compiler_params=pltpu.CompilerParams(needs_layout_passes=False),
      scratch_types=[
          pltpu.VMEM((B,), jnp.int32),
          pltpu.VMEM((BPW + L,), jnp.int32),
          pltpu.VMEM((NSLOT * G * N_NEURONS,), jnp.float32),
          pltpu.VMEM((NSLOT * G * B,), jnp.float32),
          pltpu.VMEM((BPW // 2, D_STATE), jnp.float32),
          pltpu.SemaphoreType.DMA,
          pltpu.SemaphoreType.DMA,
          pltpu.SemaphoreType.DMA,
          pltpu.SemaphoreType.DMA,
          pltpu.SemaphoreType.DMA,
          pltpu.SemaphoreType.DMA,
      ],
  )
  sigs, conns = run(indices, neuron_signatures, connection_strength)
  return sigs, conns


def kernel(indices, neuron_signatures, connection_strength):
  idx = indices.astype(jnp.int32)
  return _pool(idx, neuron_signatures, connection_strength)


# speedup vs baseline: 1.2473x; 1.0327x over previous
import jax
import jax.numpy as jnp
from jax import lax
from jax.experimental import pallas as pl
from jax.experimental.pallas import tpu as pltpu
from jax.experimental.pallas import tpu_sc as plsc

N_NEURONS = 8192
D_STATE = 256
B = 4096

NC = 2
NS = 16
L = 16
NW = NC * NS
BPW = B // NW
G = 4
NSLOT = 2
NG = BPW // G

_mesh = plsc.VectorSubcoreMesh(
    core_axis_name="c", subcore_axis_name="s", num_cores=NC, num_subcores=NS
)


def _body(idx_hbm, sig_hbm, conn_hbm, sig_out, conn_out,
          idx_all, my_idx, rows_buf, out_buf, sig_buf,
          gsem0, gsem1, osem0, osem1, ssem, osig):
  gsems = [gsem0, gsem1]
  osems = [osem0, osem1]
  cid = lax.axis_index("c")
  sid = lax.axis_index("s")
  wid = sid * NC + cid
  base = wid * BPW

  pltpu.sync_copy(idx_hbm, idx_all)
  pltpu.sync_copy(idx_hbm.at[pl.ds(base, BPW)], my_idx.at[pl.ds(0, BPW)])

  SIGC = BPW // 2
  sig_cp = pltpu.async_copy(
      sig_hbm.at[my_idx.at[pl.ds(0, SIGC)]], sig_buf, ssem
  )

  def _row_src(r):
    vec = my_idx[pl.ds(r, L)]
    return conn_hbm.at[vec[0]]

  dummy_row = conn_hbm.at[0]

  def _rslot(slot, g):
    return rows_buf.at[pl.ds((slot * G + g) * N_NEURONS, N_NEURONS)]

  def _oslot(slot, g):
    return out_buf.at[pl.ds((slot * G + g) * B, B)]

  def _start_group(gi, slot):
    for g in range(G):
      pltpu.async_copy(_row_src(gi * G + g), _rslot(slot, g), gsems[slot])

  for slot in range(NSLOT):
    _start_group(slot, slot)

  @pl.loop(0, NG // NSLOT)
  def _grp(rr):
    for slot in range(NSLOT):
      gi = rr * NSLOT + slot
      for g in range(G):
        pltpu.make_async_copy(dummy_row, _rslot(slot, g), gsems[slot]).wait()

      @pl.when(rr > 0)
      def _():
        for g in range(G):
          pltpu.make_async_copy(
              _oslot(slot, g), conn_out.at[base], osems[slot]
          ).wait()

      @pl.loop(0, B // L, unroll=8)
      def _cols(j):
        cvec = idx_all[pl.ds(j * L, L)]
        for g in range(G):
          out_buf[pl.ds((slot * G + g) * B + j * L, L)] = plsc.load_gather(
              rows_buf, [cvec + jnp.int32((slot * G + g) * N_NEURONS)]
          )

      @pl.when(gi + NSLOT < NG)
      def _():
        _start_group(gi + NSLOT, slot)

      for g in range(G):
        pltpu.async_copy(
            _oslot(slot, g), conn_out.at[base + gi * G + g], osems[slot]
        )

  for slot in range(NSLOT):
    for g in range(G):
      pltpu.make_async_copy(
          _oslot(slot, g), conn_out.at[base], osems[slot]
      ).wait()

  sig_cp.wait()
  pltpu.async_copy(sig_buf, sig_out.at[pl.ds(base, SIGC)], osig).wait()
  pltpu.async_copy(
      sig_hbm.at[my_idx.at[pl.ds(SIGC, SIGC)]], sig_buf, ssem
  ).wait()
  pltpu.sync_copy(sig_buf, sig_out.at[pl.ds(base + SIGC, SIGC)])


@jax.jit
def _pool(indices, neuron_signatures, connection_strength):
  run = pl.kernel(
      _body,
      out_type=[
          jax.ShapeDtypeStruct((B, D_STATE), jnp.float32),
          jax.ShapeDtypeStruct((B, B), jnp.float32),
      ],
      mesh=_mesh,
      compiler_params=pltpu.CompilerParams(needs_layout_passes=False),
      scratch_types=[
          pltpu.VMEM((B,), jnp.int32),
          pltpu.VMEM((BPW + L,), jnp.int32),
          pltpu.VMEM((NSLOT * G * N_NEURONS,), jnp.float32),
          pltpu.VMEM((NSLOT * G * B,), jnp.float32),
          pltpu.VMEM((BPW // 2, D_STATE), jnp.float32),
          pltpu.SemaphoreType.DMA,
          pltpu.SemaphoreType.DMA,
          pltpu.SemaphoreType.DMA,
          pltpu.SemaphoreType.DMA,
          pltpu.SemaphoreType.DMA,
          pltpu.SemaphoreType.DMA,
      ],
  )
  sigs, conns = run(indices, neuron_signatures, connection_strength)
  return sigs, conns


def kernel(indices, neuron_signatures, connection_strength):
  idx = indices.astype(jnp.int32)
  return _pool(idx, neuron_signatures, connection_strength)
